# baseline (speedup 1.0000x reference)
import dataclasses
import functools

import jax
import jax.numpy as jnp
from jax import lax
from jax.experimental import pallas as pl
from jax.experimental.pallas import tpu as pltpu
from jax.experimental.pallas import tpu_sc as plsc

TEMP = 0.07
NUM_ROWS = 22
NJ = NUM_ROWS - 1
NC, NS = 2, 16
NW = NC * NS
CB = 128
LANES = 16
DCHUNKS = 128 // LANES
RING = 3


def _sc_scores(embeddings, idx_r, batch, dim, n_chunks):
    mesh = plsc.VectorSubcoreMesh(
        core_axis_name="c", subcore_axis_name="s", num_cores=NC, num_subcores=NS
    )
    cp = pltpu.CompilerParams()
    if "needs_layout_passes" in pltpu.CompilerParams.__dataclass_fields__:
        cp = dataclasses.replace(cp, needs_layout_passes=False)

    @functools.partial(
        pl.kernel,
        compiler_params=cp,
        out_type=jax.ShapeDtypeStruct((batch, NJ * LANES), jnp.float32),
        mesh=mesh,
        scratch_types=[
            pltpu.VMEM((NUM_ROWS, CB), jnp.int32),
            pltpu.VMEM((CB, dim), jnp.float32),
            pltpu.VMEM((CB, dim), jnp.float32),
            pltpu.VMEM((CB, dim), jnp.float32),
            pltpu.VMEM((CB, dim), jnp.float32),
            pltpu.VMEM((CB, NJ * LANES), jnp.float32),
            pltpu.SemaphoreType.DMA,
            pltpu.SemaphoreType.DMA,
            pltpu.SemaphoreType.DMA,
            pltpu.SemaphoreType.DMA,
        ],
    )
    def k(emb_hbm, idx_hbm, out_hbm, idx_v, tbuf, nb0, nb1, nb2,
          scores, semt, sem0, sem1, sem2):
        wid = lax.axis_index("s") * NC + lax.axis_index("c")
        nbuf = (nb0, nb1, nb2)
        sem = (sem0, sem1, sem2)

        @pl.loop(0, n_chunks)
        def _(c):
            base = wid * (n_chunks * CB) + c * CB
            pltpu.sync_copy(idx_hbm.at[wid, c], idx_v)
            pltpu.async_copy(emb_hbm.at[idx_v.at[0]], tbuf, semt)
            for j in range(1, 1 + RING):
                s = (j - 1) % RING
                pltpu.async_copy(emb_hbm.at[idx_v.at[j]], nbuf[s], sem[s])
            pltpu.make_async_copy(emb_hbm.at[idx_v.at[0]], tbuf, semt).wait()
            for j0 in range(1, NUM_ROWS, 2):
                js = [j for j in (j0, j0 + 1) if j < NUM_ROWS]
                for j in js:
                    s = (j - 1) % RING
                    pltpu.make_async_copy(emb_hbm.at[idx_v.at[j]],
                                          nbuf[s], sem[s]).wait()

                @plsc.parallel_loop(0, CB, unroll=2)
                def _(i):
                    t = [tbuf[i, pl.ds(kk * LANES, LANES)]
                         for kk in range(DCHUNKS)]
                    for j in js:
                        nb = nbuf[(j - 1) % RING]
                        acc = t[0] * nb[i, pl.ds(0, LANES)]
                        for kk in range(1, DCHUNKS):
                            acc = acc + (t[kk]
                                         * nb[i, pl.ds(kk * LANES, LANES)])
                        scores[i, pl.ds((j - 1) * LANES, LANES)] = acc

                for j in js:
                    if j + RING < NUM_ROWS:
                        pltpu.async_copy(emb_hbm.at[idx_v.at[j + RING]],
                                         nbuf[(j - 1) % RING],
                                         sem[(j - 1) % RING])

            pltpu.sync_copy(scores, out_hbm.at[pl.ds(base, CB), :])

    return k(embeddings, idx_r)


def _tc_loss(scores, batch):
    bt = 2048
    nblk = batch // bt

    def body(s_ref, out_ref):
        i = pl.program_id(0)
        p = s_ref[...]
        r = lax.broadcasted_iota(jnp.int32, (NJ * LANES, NJ), 0)
        c = lax.broadcasted_iota(jnp.int32, (NJ * LANES, NJ), 1)
        w = (r // LANES == c).astype(jnp.float32)
        s = jnp.dot(p, w, preferred_element_type=jnp.float32) * (1.0 / TEMP)
        m = jnp.max(s, axis=1)
        lse = jnp.log(jnp.sum(jnp.exp(s - m[:, None]), axis=1)) + m
        part = jnp.sum(lse - s[:, 0])

        @pl.when(i == 0)
        def _():
            out_ref[0, 0] = 0.0

        acc = out_ref[0, 0] + part
        out_ref[0, 0] = jnp.where(i == nblk - 1, acc / batch, acc)

    out = pl.pallas_call(
        body,
        grid=(nblk,),
        in_specs=[pl.BlockSpec((bt, NJ * LANES), lambda i: (i, 0))],
        out_specs=pl.BlockSpec(
            (1, 1), lambda i: (0, 0), memory_space=pltpu.SMEM
        ),
        out_shape=jax.ShapeDtypeStruct((1, 1), jnp.float32),
    )(scores)
    return out[0, 0]


def kernel(embeddings, targets, contexts, negatives):
    batch, num_neg = negatives.shape
    dim = embeddings.shape[1]
    n_chunks = batch // (NW * CB)
    idx_all = jnp.concatenate(
        [targets[None, :], contexts[None, :], negatives.T], axis=0
    ).astype(jnp.int32)
    idx_r = (
        idx_all.reshape(NUM_ROWS, NW, n_chunks, CB).transpose(1, 2, 0, 3)
    )
    scores = _sc_scores(embeddings, idx_r, batch, dim, n_chunks)
    return _tc_loss(scores, batch)


# speedup vs baseline: 1.3402x; 1.3402x over previous
import dataclasses
import functools

import jax
import jax.numpy as jnp
from jax import lax
from jax.experimental import pallas as pl
from jax.experimental.pallas import tpu as pltpu
from jax.experimental.pallas import tpu_sc as plsc

TEMP = 0.07
NUM_ROWS = 22
NJ = NUM_ROWS - 1
NC, NS = 2, 16
NW = NC * NS
CB = 128
LANES = 16
DCHUNKS = 128 // LANES
RING = 3


def _sc_scores(embeddings, idx_r, batch, dim, n_chunks):
    mesh = plsc.VectorSubcoreMesh(
        core_axis_name="c", subcore_axis_name="s", num_cores=NC, num_subcores=NS
    )
    cp = pltpu.CompilerParams()
    if "needs_layout_passes" in pltpu.CompilerParams.__dataclass_fields__:
        cp = dataclasses.replace(cp, needs_layout_passes=False)

    @functools.partial(
        pl.kernel,
        compiler_params=cp,
        out_type=jax.ShapeDtypeStruct((batch, NJ * LANES), jnp.float32),
        mesh=mesh,
        scratch_types=[
            pltpu.VMEM((NUM_ROWS, CB), jnp.int32),
            pltpu.VMEM((CB, dim), jnp.float32),
            pltpu.VMEM((CB, dim), jnp.float32),
            pltpu.VMEM((CB, dim), jnp.float32),
            pltpu.VMEM((CB, dim), jnp.float32),
            pltpu.VMEM((CB, NJ * LANES), jnp.float32),
            pltpu.SemaphoreType.DMA,
            pltpu.SemaphoreType.DMA,
            pltpu.SemaphoreType.DMA,
            pltpu.SemaphoreType.DMA,
        ],
    )
    def k(emb_hbm, idx_hbm, out_hbm, idx_v, tbuf, nb0, nb1, nb2,
          scores, semt, sem0, sem1, sem2):
        wid = lax.axis_index("s") * NC + lax.axis_index("c")
        nbuf = (nb0, nb1, nb2)
        sem = (sem0, sem1, sem2)

        @pl.loop(0, n_chunks)
        def _(c):
            base = wid * (n_chunks * CB) + c * CB
            pltpu.sync_copy(idx_hbm.at[wid, c], idx_v)
            pltpu.async_copy(emb_hbm.at[idx_v.at[0]], tbuf, semt)
            for j in range(1, 1 + RING):
                s = (j - 1) % RING
                pltpu.async_copy(emb_hbm.at[idx_v.at[j]], nbuf[s], sem[s])
            pltpu.make_async_copy(emb_hbm.at[idx_v.at[0]], tbuf, semt).wait()
            for j in range(1, NUM_ROWS):
                s = (j - 1) % RING
                nb = nbuf[s]
                pltpu.make_async_copy(emb_hbm.at[idx_v.at[j]], nb,
                                      sem[s]).wait()

                @plsc.parallel_loop(0, CB, unroll=2)
                def _(i):
                    acc = (tbuf[i, pl.ds(0, LANES)]
                           * nb[i, pl.ds(0, LANES)])
                    for kk in range(1, DCHUNKS):
                        acc = acc + (tbuf[i, pl.ds(kk * LANES, LANES)]
                                     * nb[i, pl.ds(kk * LANES, LANES)])
                    scores[i, pl.ds((j - 1) * LANES, LANES)] = acc

                if j + RING < NUM_ROWS:
                    pltpu.async_copy(emb_hbm.at[idx_v.at[j + RING]],
                                     nbuf[s], sem[s])

            pltpu.sync_copy(scores, out_hbm.at[pl.ds(base, CB), :])

    return k(embeddings, idx_r)


def _tc_loss(scores, batch):
    bt = 4096
    nblk = batch // bt

    def body(s_ref, out_ref):
        i = pl.program_id(0)
        p = s_ref[...]
        r = lax.broadcasted_iota(jnp.int32, (NJ * LANES, NJ), 0)
        c = lax.broadcasted_iota(jnp.int32, (NJ * LANES, NJ), 1)
        w = (r // LANES == c).astype(jnp.float32)
        s = jnp.dot(p, w, preferred_element_type=jnp.float32) * (1.0 / TEMP)
        m = jnp.max(s, axis=1)
        lse = jnp.log(jnp.sum(jnp.exp(s - m[:, None]), axis=1)) + m
        part = jnp.sum(lse - s[:, 0])

        @pl.when(i == 0)
        def _():
            out_ref[0, 0] = 0.0

        acc = out_ref[0, 0] + part
        out_ref[0, 0] = jnp.where(i == nblk - 1, acc / batch, acc)

    out = pl.pallas_call(
        body,
        grid=(nblk,),
        in_specs=[pl.BlockSpec((bt, NJ * LANES), lambda i: (i, 0))],
        out_specs=pl.BlockSpec(
            (1, 1), lambda i: (0, 0), memory_space=pltpu.SMEM
        ),
        out_shape=jax.ShapeDtypeStruct((1, 1), jnp.float32),
    )(scores)
    return out[0, 0]


def kernel(embeddings, targets, contexts, negatives):
    batch, num_neg = negatives.shape
    dim = embeddings.shape[1]
    n_chunks = batch // (NW * CB)
    idx_all = jnp.concatenate(
        [targets[None, :], contexts[None, :], negatives.T], axis=0
    ).astype(jnp.int32)
    idx_r = (
        idx_all.reshape(NUM_ROWS, NW, n_chunks, CB).transpose(1, 2, 0, 3)
    )
    scores = _sc_scores(embeddings, idx_r, batch, dim, n_chunks)
    return _tc_loss(scores, batch)
